# Initial kernel scaffold; baseline (speedup 1.0000x reference)
import jax
import jax.numpy as jnp
from jax.experimental import pallas as pl


def kernel(token_ids, field_pos, header_pos, token_embed, token_pos_embed, field_pos_embed, header_pos_embed):
    raise NotImplementedError("write your pallas kernel here")


# speedup vs baseline: 2.0503x; 2.0503x over previous
import functools

import jax
import jax.numpy as jnp
from jax import lax
from jax.experimental import pallas as pl
from jax.experimental.pallas import tpu as pltpu
from jax.experimental.pallas import tpu_sc as plsc

VOCAB = 1000000
MAX_LEN = 200
EMBED = 32
B = 16384
L = 50
N = B * L

NUM_CORES = 2
NUM_SUBCORES = 16
NW = NUM_CORES * NUM_SUBCORES
ROWS_W = N // NW
CHUNK = 640
NCHUNK = ROWS_W // CHUNK
SUB = 128
NSUB = CHUNK // SUB
NGROUP = CHUNK // 16

_mesh = plsc.VectorSubcoreMesh(core_axis_name="c", subcore_axis_name="s")


@functools.partial(
    pl.kernel,
    out_type=jax.ShapeDtypeStruct((N, EMBED), jnp.float32),
    mesh=_mesh,
    compiler_params=pltpu.CompilerParams(needs_layout_passes=False,
                                         use_tc_tiling_on_sc=False),
    scratch_types=[
        pltpu.VMEM((CHUNK,), jnp.int32),
        pltpu.VMEM((CHUNK,), jnp.int32),
        pltpu.VMEM((CHUNK,), jnp.int32),
        pltpu.VMEM((CHUNK, EMBED), jnp.float32),
        pltpu.VMEM((MAX_LEN, EMBED), jnp.float32),
        pltpu.VMEM((MAX_LEN, EMBED), jnp.float32),
        pltpu.VMEM((MAX_LEN, EMBED), jnp.float32),
        pltpu.SemaphoreType.DMA,
    ],
)
def _packet_embed(tok, fld, hdr, temb, tpe, fpe, hpe, out,
                  tok_v, fld_v, hdr_v, buf_v, tpe_v, fpe_v, hpe_v, sem):
    wid = lax.axis_index("s") * NUM_CORES + lax.axis_index("c")
    base_w = wid * ROWS_W

    pltpu.sync_copy(tpe, tpe_v)
    pltpu.sync_copy(fpe, fpe_v)
    pltpu.sync_copy(hpe, hpe_v)

    def chunk_body(ci, carry):
        base = base_w + ci * CHUNK
        pltpu.sync_copy(tok.at[pl.ds(base, CHUNK)], tok_v)
        pltpu.sync_copy(fld.at[pl.ds(base, CHUNK)], fld_v)
        pltpu.sync_copy(hdr.at[pl.ds(base, CHUNK)], hdr_v)
        cps = [
            pltpu.async_copy(
                temb.at[tok_v.at[pl.ds(j * SUB, SUB)]],
                buf_v.at[pl.ds(j * SUB, SUB)],
                sem,
            )
            for j in range(NSUB)
        ]
        for cp in cps:
            cp.wait()

        def group_body(g, inner):
            r0 = g * 16
            rows = r0 + lax.iota(jnp.int32, 16)
            lidx = lax.rem(base + rows, L)
            fi = fld_v[pl.ds(r0, 16)]
            hi = hdr_v[pl.ds(r0, 16)]
            for d in range(EMBED):
                dv = jnp.full((16,), d, jnp.int32)
                v = (plsc.load_gather(tpe_v, [lidx, dv])
                     + plsc.load_gather(fpe_v, [fi, dv])
                     + plsc.load_gather(hpe_v, [hi, dv]))
                plsc.addupdate_scatter(buf_v, [rows, dv], v)
            return inner

        lax.fori_loop(0, NGROUP, group_body, 0)
        pltpu.sync_copy(buf_v, out.at[pl.ds(base, CHUNK)])
        return carry

    lax.fori_loop(0, NCHUNK, chunk_body, 0)


def kernel(token_ids, field_pos, header_pos, token_embed, token_pos_embed,
           field_pos_embed, header_pos_embed):
    tok = jnp.reshape(token_ids, (N,)).astype(jnp.int32)
    fld = jnp.reshape(field_pos, (N,)).astype(jnp.int32)
    hdr = jnp.reshape(header_pos, (N,)).astype(jnp.int32)
    out = _packet_embed(tok, fld, hdr, token_embed, token_pos_embed,
                        field_pos_embed, header_pos_embed)
    return jnp.reshape(out, (B, L, EMBED))


# speedup vs baseline: 3.7960x; 1.8515x over previous
import functools

import jax
import jax.numpy as jnp
from jax import lax
from jax.experimental import pallas as pl
from jax.experimental.pallas import tpu as pltpu
from jax.experimental.pallas import tpu_sc as plsc

VOCAB = 1000000
MAX_LEN = 200
EMBED = 32
B = 16384
L = 50
N = B * L

NUM_CORES = 2
NUM_SUBCORES = 16
NW = NUM_CORES * NUM_SUBCORES
ROWS_W = N // NW
CHUNK = 640
NCHUNK = ROWS_W // CHUNK
SUB = 128
NSUB = CHUNK // SUB
NBUF = 4
NITER = NCHUNK // NBUF
RUNROLL = 4

_mesh = plsc.VectorSubcoreMesh(core_axis_name="c", subcore_axis_name="s")


@functools.partial(
    pl.kernel,
    out_type=jax.ShapeDtypeStruct((N, EMBED), jnp.float32),
    mesh=_mesh,
    compiler_params=pltpu.CompilerParams(needs_layout_passes=False,
                                         use_tc_tiling_on_sc=False),
    scratch_types=[
        pltpu.VMEM((NBUF, CHUNK), jnp.int32),
        pltpu.VMEM((NBUF, CHUNK), jnp.int32),
        pltpu.VMEM((NBUF, CHUNK), jnp.int32),
        pltpu.VMEM((NBUF, CHUNK, EMBED), jnp.float32),
        pltpu.VMEM((MAX_LEN, EMBED), jnp.float32),
        pltpu.VMEM((MAX_LEN, EMBED), jnp.float32),
        pltpu.VMEM((MAX_LEN, EMBED), jnp.float32),
        [pltpu.SemaphoreType.DMA] * NBUF,
        [pltpu.SemaphoreType.DMA] * NBUF,
    ],
)
def _packet_embed(tok, fld, hdr, temb, tpe, fpe, hpe, out,
                  tok_v, fld_v, hdr_v, buf_v, tpe_v, fpe_v, hpe_v,
                  gsems, osems):
    wid = lax.axis_index("s") * NUM_CORES + lax.axis_index("c")
    base_w = wid * ROWS_W

    pltpu.sync_copy(tpe, tpe_v)
    pltpu.sync_copy(fpe, fpe_v)
    pltpu.sync_copy(hpe, hpe_v)

    def stage(c, s):
        base = base_w + c * CHUNK
        pltpu.sync_copy(tok.at[pl.ds(base, CHUNK)], tok_v.at[s])
        pltpu.sync_copy(fld.at[pl.ds(base, CHUNK)], fld_v.at[s])
        pltpu.sync_copy(hdr.at[pl.ds(base, CHUNK)], hdr_v.at[s])
        for j in range(NSUB):
            pltpu.async_copy(
                temb.at[tok_v.at[s, pl.ds(j * SUB, SUB)]],
                buf_v.at[s, pl.ds(j * SUB, SUB)],
                gsems[s],
            )

    def drain_gathers(s):
        for j in range(NSUB):
            pltpu.make_async_copy(
                temb.at[tok_v.at[s, pl.ds(j * SUB, SUB)]],
                buf_v.at[s, pl.ds(j * SUB, SUB)],
                gsems[s],
            ).wait()

    def wait_out(c, s):
        base = base_w + c * CHUNK
        pltpu.make_async_copy(buf_v.at[s], out.at[pl.ds(base, CHUNK)],
                              osems[s]).wait()

    def compute(c, s):
        base = base_w + c * CHUNK

        def row_block(rb, inner):
            r0 = rb * 16
            fvec = fld_v[s, pl.ds(r0, 16)]
            hvec = hdr_v[s, pl.ds(r0, 16)]
            for k in range(16):
                r = r0 + k
                f = fvec[k]
                h = hvec[k]
                l = lax.rem(base + r, L)
                for half in (0, 16):
                    sl = pl.ds(half, 16)
                    acc = (buf_v[s, r, sl] + tpe_v[l, sl]
                           + fpe_v[f, sl] + hpe_v[h, sl])
                    buf_v[s, r, sl] = acc
            return inner

        lax.fori_loop(0, CHUNK // 16, row_block, 0)

    stage(0, 0)

    def iter_body(k, carry):
        ii = k * NBUF
        for j in range(NBUF):
            c = ii + j
            sn = (j + 1) % NBUF

            @pl.when(jnp.logical_and(c >= NBUF - 1, c + 1 < NCHUNK))
            def _():
                wait_out(c + 1 - NBUF, sn)

            @pl.when(c + 1 < NCHUNK)
            def _():
                stage(c + 1, sn)

            drain_gathers(j)
            compute(c, j)
            pltpu.async_copy(buf_v.at[j],
                             out.at[pl.ds(base_w + c * CHUNK, CHUNK)],
                             osems[j])
        return carry

    lax.fori_loop(0, NITER, iter_body, 0)

    for j in range(NBUF):
        wait_out(NCHUNK - NBUF + j, j)


def kernel(token_ids, field_pos, header_pos, token_embed, token_pos_embed,
           field_pos_embed, header_pos_embed):
    tok = jnp.reshape(token_ids, (N,)).astype(jnp.int32)
    fld = jnp.reshape(field_pos, (N,)).astype(jnp.int32)
    hdr = jnp.reshape(header_pos, (N,)).astype(jnp.int32)
    out = _packet_embed(tok, fld, hdr, token_embed, token_pos_embed,
                        field_pos_embed, header_pos_embed)
    return jnp.reshape(out, (B, L, EMBED))


# speedup vs baseline: 5.3892x; 1.4197x over previous
import functools

import jax
import jax.numpy as jnp
from jax import lax
from jax.experimental import pallas as pl
from jax.experimental.pallas import tpu as pltpu
from jax.experimental.pallas import tpu_sc as plsc

VOCAB = 1000000
MAX_LEN = 200
EMBED = 32
B = 16384
L = 50
N = B * L

NUM_CORES = 2
NUM_SUBCORES = 16
NW = NUM_CORES * NUM_SUBCORES
ROWS_W = N // NW
CHUNK = 400
PKCHUNK = CHUNK // L
NCHUNK = ROWS_W // CHUNK
SUBS = (104, 104, 104, 88)
NBUF = 4
NITER = NCHUNK // NBUF

_mesh = plsc.VectorSubcoreMesh(core_axis_name="c", subcore_axis_name="s")


@functools.partial(
    pl.kernel,
    out_type=jax.ShapeDtypeStruct((B, L, EMBED), jnp.float32),
    mesh=_mesh,
    compiler_params=pltpu.CompilerParams(needs_layout_passes=False,
                                         use_tc_tiling_on_sc=False),
    scratch_types=[
        pltpu.VMEM((NBUF, CHUNK), jnp.int32),
        pltpu.VMEM((NBUF, CHUNK), jnp.int32),
        pltpu.VMEM((NBUF, CHUNK), jnp.int32),
        pltpu.VMEM((NBUF, CHUNK, EMBED), jnp.float32),
        pltpu.VMEM((MAX_LEN, EMBED), jnp.float32),
        pltpu.VMEM((MAX_LEN, EMBED), jnp.float32),
        pltpu.VMEM((MAX_LEN, EMBED), jnp.float32),
        [pltpu.SemaphoreType.DMA] * NBUF,
        [pltpu.SemaphoreType.DMA] * NBUF,
    ],
)
def _packet_embed(tok, fld, hdr, temb, tpe, fpe, hpe, out,
                  tok_v, fld_v, hdr_v, buf_v, tpe_v, fpe_v, hpe_v,
                  gsems, osems):
    wid = lax.axis_index("s") * NUM_CORES + lax.axis_index("c")
    base_w = wid * ROWS_W
    pk_w = base_w // L

    pltpu.sync_copy(tpe, tpe_v)
    pltpu.sync_copy(fpe, fpe_v)
    pltpu.sync_copy(hpe, hpe_v)

    def stage(c, s):
        base = base_w + c * CHUNK
        pltpu.sync_copy(tok.at[pl.ds(base, CHUNK)], tok_v.at[s])
        pltpu.sync_copy(fld.at[pl.ds(base, CHUNK)], fld_v.at[s])
        pltpu.sync_copy(hdr.at[pl.ds(base, CHUNK)], hdr_v.at[s])
        off = 0
        for sub in SUBS:
            pltpu.async_copy(
                temb.at[tok_v.at[s, pl.ds(off, sub)]],
                buf_v.at[s, pl.ds(off, sub), :],
                gsems[s],
            )
            off += sub

    def drain_gathers(s):
        off = 0
        for sub in SUBS:
            pltpu.make_async_copy(
                temb.at[tok_v.at[s, pl.ds(off, sub)]],
                buf_v.at[s, pl.ds(off, sub), :],
                gsems[s],
            ).wait()
            off += sub

    def fire_out(c, s):
        pk = pk_w + c * PKCHUNK
        for p in range(PKCHUNK):
            pltpu.async_copy(buf_v.at[s, pl.ds(p * L, L), :],
                             out.at[pk + p], osems[s])

    def wait_out(c, s):
        pk = pk_w + c * PKCHUNK
        for p in range(PKCHUNK):
            pltpu.make_async_copy(buf_v.at[s, pl.ds(p * L, L), :],
                                  out.at[pk + p], osems[s]).wait()

    def compute(c, s):
        base = base_w + c * CHUNK

        def group_body(g, inner):
            r0 = g * 16
            fvec = fld_v[s, pl.ds(r0, 16)]
            hvec = hdr_v[s, pl.ds(r0, 16)]
            for k in range(16):
                r = r0 + k
                f = fvec[k]
                h = hvec[k]
                l = lax.rem(base + r, L)
                for half in (0, 16):
                    sl = pl.ds(half, 16)
                    acc = (buf_v[s, r, sl] + tpe_v[l, sl]
                           + fpe_v[f, sl] + hpe_v[h, sl])
                    buf_v[s, r, sl] = acc
            return inner

        lax.fori_loop(0, CHUNK // 16, group_body, 0)

    stage(0, 0)

    def iter_body(k, carry):
        ii = k * NBUF
        for j in range(NBUF):
            c = ii + j
            sn = (j + 1) % NBUF

            @pl.when(jnp.logical_and(c >= NBUF - 1, c + 1 < NCHUNK))
            def _():
                wait_out(c + 1 - NBUF, sn)

            @pl.when(c + 1 < NCHUNK)
            def _():
                stage(c + 1, sn)

            drain_gathers(j)
            compute(c, j)
            fire_out(c, j)
        return carry

    lax.fori_loop(0, NITER, iter_body, 0)

    for j in range(NBUF):
        wait_out(NCHUNK - NBUF + j, j)


def kernel(token_ids, field_pos, header_pos, token_embed, token_pos_embed,
           field_pos_embed, header_pos_embed):
    tok = jnp.reshape(token_ids, (N,)).astype(jnp.int32)
    fld = jnp.reshape(field_pos, (N,)).astype(jnp.int32)
    hdr = jnp.reshape(header_pos, (N,)).astype(jnp.int32)
    return _packet_embed(tok, fld, hdr, token_embed, token_pos_embed,
                         field_pos_embed, header_pos_embed)
